# baseline (speedup 1.0000x reference)
import jax
import jax.numpy as jnp
from jax import lax
from jax.experimental import pallas as pl
from jax.experimental.pallas import tpu as pltpu
from jax.experimental.pallas import tpu_sc as plsc

N = 10000
NUM_NBR = 32
E = N * NUM_NBR
ATOM = 128
NBR = 16
C = 2 * ATOM + NBR
CW = 256
CP = 2 * CW
GW = 144
HMASK = -65536
RHALF = 0x8000

NC = 2
NS = 16
NW = NC * NS
EPW = E // NW
CB = 40
NCHUNK = EPW // CB

BE = 6400
NBLK = E // BE
BN_ROWS = BE // NUM_NBR
XBLK = 10000


def _softplus(v):
    return jnp.maximum(v, 0.0) + jnp.log1p(jnp.exp(-jnp.abs(v)))


def _sigmoid(v):
    return 1.0 / (1.0 + jnp.exp(-v))


def _node_transform_body(x_ref, wd_ref, ws_ref, p_ref, q_ref):
    xv = x_ref[...]
    p_ref[...] = jnp.dot(xv, wd_ref[...],
                         preferred_element_type=jnp.float32).astype(jnp.bfloat16)
    q_ref[...] = jnp.dot(xv, ws_ref[...],
                         preferred_element_type=jnp.float32).astype(jnp.bfloat16)


def _node_transform(x, wdt, wst):
    return pl.pallas_call(
        _node_transform_body,
        grid=(N // XBLK,),
        in_specs=[
            pl.BlockSpec((XBLK, ATOM), lambda i: (i, 0)),
            pl.BlockSpec((ATOM, CP), lambda i: (0, 0)),
            pl.BlockSpec((ATOM, CP), lambda i: (0, 0)),
        ],
        out_specs=[
            pl.BlockSpec((XBLK, CP), lambda i: (i, 0)),
            pl.BlockSpec((XBLK, CP), lambda i: (i, 0)),
        ],
        out_shape=[jax.ShapeDtypeStruct((N, CP), jnp.bfloat16)] * 2,
    )(x, wdt, wst)


def _sc_gather_body(p_hbm, q_hbm, didx_hbm, sidx_hbm, g_hbm,
                    idxd_v, idxs_v, bufp, bufq, bufg,
                    semp0, semp1, semq0, semq1, semw0, semw1):
    wid = lax.axis_index("s") * NC + lax.axis_index("c")
    base = pl.multiple_of(wid * EPW, 8)
    pltpu.sync_copy(didx_hbm.at[pl.ds(base, EPW)], idxd_v)
    pltpu.sync_copy(sidx_hbm.at[pl.ds(base, EPW)], idxs_v)
    semp = (semp0, semp1)
    semq = (semq0, semq1)
    semw = (semw0, semw1)

    def fire(c, b):
        coff = pl.multiple_of(c * CB, 8)
        pltpu.async_copy(p_hbm.at[idxd_v.at[pl.ds(coff, CB)]],
                         bufp.at[b], semp[b])
        pltpu.async_copy(q_hbm.at[idxs_v.at[pl.ds(coff, CB)]],
                         bufq.at[b], semq[b])

    def wait_gathers(b):
        pltpu.make_async_copy(p_hbm.at[idxd_v.at[pl.ds(0, CB)]],
                              bufp.at[b], semp[b]).wait()
        pltpu.make_async_copy(q_hbm.at[idxs_v.at[pl.ds(0, CB)]],
                              bufq.at[b], semq[b]).wait()

    def wait_write(b):
        pltpu.make_async_copy(bufg.at[b], g_hbm.at[pl.ds(0, CB)],
                              semw[b]).wait()

    fire(0, 0)

    def outer(ci2, carry):
        for b in range(2):
            c = ci2 * 2 + b
            nxt_ok = c + 1 < NCHUNK

            @pl.when(jnp.logical_and(nxt_ok, c >= 1))
            def _():
                wait_write(1 - b)

            @pl.when(nxt_ok)
            def _():
                fire(c + 1, 1 - b)

            wait_gathers(b)

            def row(r, cc):
                for j in range(GW // 16):
                    sl = pl.ds(j * 16, 16)
                    wp = bufp[b, r, sl]
                    wq = bufq[b, r, sl]
                    bcf = lambda v: lax.bitcast_convert_type(v, jnp.float32)
                    bci = lambda v: lax.bitcast_convert_type(v, jnp.int32)
                    lo = (bcf(lax.shift_left(wp, 16))
                          + bcf(lax.shift_left(wq, 16)))
                    lor = lax.shift_right_logical(bci(lo) + RHALF, 16)
                    hi = bcf(wp & HMASK) + bcf(wq & HMASK)
                    hir = (bci(hi) + RHALF) & HMASK
                    bufg[b, r, sl] = hir | lor
                return cc

            lax.fori_loop(0, CB, row, 0)
            off = pl.multiple_of(base + c * CB, 8)
            pltpu.async_copy(bufg.at[b], g_hbm.at[pl.ds(off, CB)], semw[b])
        return carry

    lax.fori_loop(0, NCHUNK // 2, outer, 0)
    wait_write(0)
    wait_write(1)


def _sc_gather(p, q, didx, sidx):
    mesh = plsc.VectorSubcoreMesh(core_axis_name="c", subcore_axis_name="s")
    fn = pl.kernel(
        _sc_gather_body,
        mesh=mesh,
        out_type=jax.ShapeDtypeStruct((E, GW), jnp.int32),
        scratch_types=[
            pltpu.VMEM((EPW,), jnp.int32),
            pltpu.VMEM((EPW,), jnp.int32),
            pltpu.VMEM((2, CB, CW), jnp.int32),
            pltpu.VMEM((2, CB, CW), jnp.int32),
            pltpu.VMEM((2, CB, GW), jnp.int32),
            pltpu.SemaphoreType.DMA,
            pltpu.SemaphoreType.DMA,
            pltpu.SemaphoreType.DMA,
            pltpu.SemaphoreType.DMA,
            pltpu.SemaphoreType.DMA,
            pltpu.SemaphoreType.DMA,
        ],
    )
    return fn(p, q, didx, sidx)


def _unpack(gi):
    f_lo = lax.bitcast_convert_type(lax.shift_left(gi, 16), jnp.float32)
    f_hi = lax.bitcast_convert_type(gi & HMASK, jnp.float32)
    return f_lo, f_hi


def _heads(gi, a, wcor_ref, wfil_ref, wbnd_ref):
    f_lo, f_hi = _unpack(gi)
    h_core = f_lo[:, :ATOM] + jnp.dot(a, wcor_ref[...],
                                      preferred_element_type=jnp.float32)
    h_filt = f_hi[:, :ATOM] + jnp.dot(a, wfil_ref[...],
                                      preferred_element_type=jnp.float32)
    h_bond = (jnp.concatenate([f_lo[:, ATOM:ATOM + 8],
                               f_hi[:, ATOM:ATOM + 8]], axis=1)
              + jnp.dot(a, wbnd_ref[...], preferred_element_type=jnp.float32))
    return h_core, h_filt, h_bond


def _stats_body(g_ref, a_ref, wcor_ref, wfil_ref, wbnd_ref,
                sc_ref, qc_ref, sf_ref, qf_ref, sb_ref, qb_ref):
    h_core, h_filt, h_bond = _heads(g_ref[...], a_ref[...],
                                    wcor_ref, wfil_ref, wbnd_ref)

    @pl.when(pl.program_id(0) == 0)
    def _():
        for r in (sc_ref, qc_ref, sf_ref, qf_ref, sb_ref, qb_ref):
            r[...] = jnp.zeros_like(r)

    sc_ref[...] += jnp.sum(h_core, axis=0, keepdims=True)
    qc_ref[...] += jnp.sum(h_core * h_core, axis=0, keepdims=True)
    sf_ref[...] += jnp.sum(h_filt, axis=0, keepdims=True)
    qf_ref[...] += jnp.sum(h_filt * h_filt, axis=0, keepdims=True)
    sb_ref[...] += jnp.sum(h_bond, axis=0, keepdims=True)
    qb_ref[...] += jnp.sum(h_bond * h_bond, axis=0, keepdims=True)


def _stats(g, edge_attr, wcor, wfil, wbnd):
    return pl.pallas_call(
        _stats_body,
        grid=(NBLK,),
        in_specs=[
            pl.BlockSpec((BE, GW), lambda i: (i, 0)),
            pl.BlockSpec((BE, NBR), lambda i: (i, 0)),
            pl.BlockSpec((NBR, ATOM), lambda i: (0, 0)),
            pl.BlockSpec((NBR, ATOM), lambda i: (0, 0)),
            pl.BlockSpec((NBR, NBR), lambda i: (0, 0)),
        ],
        out_specs=[pl.BlockSpec((1, ATOM), lambda i: (0, 0))] * 4
        + [pl.BlockSpec((1, NBR), lambda i: (0, 0))] * 2,
        out_shape=[jax.ShapeDtypeStruct((1, ATOM), jnp.float32)] * 4
        + [jax.ShapeDtypeStruct((1, NBR), jnp.float32)] * 2,
    )(g, edge_attr, wcor, wfil, wbnd)


def _main_body(g_ref, a_ref, wcor_ref, wfil_ref, wbnd_ref,
               scc_ref, shc_ref, scf_ref, shf_ref, scb_ref, shb_ref,
               nbr_ref, bond_ref, s_ref, sq_ref):
    a = a_ref[...]
    h_core, h_filt, h_bond = _heads(g_ref[...], a,
                                    wcor_ref, wfil_ref, wbnd_ref)
    af = _sigmoid(h_core * scc_ref[...] + shc_ref[...])
    ac = _softplus(h_filt * scf_ref[...] + shf_ref[...])
    prod = af * ac
    nb = prod.reshape(BN_ROWS, NUM_NBR, ATOM).sum(axis=1)
    nbr_ref[...] = nb
    bond_ref[...] = _softplus(a + (h_bond * scb_ref[...] + shb_ref[...]))

    @pl.when(pl.program_id(0) == 0)
    def _():
        s_ref[...] = jnp.zeros_like(s_ref)
        sq_ref[...] = jnp.zeros_like(sq_ref)

    s_ref[...] += jnp.sum(nb, axis=0, keepdims=True)
    sq_ref[...] += jnp.sum(nb * nb, axis=0, keepdims=True)


def _main(g, edge_attr, wcor, wfil, wbnd, scc, shc, scf, shf, scb, shb):
    return pl.pallas_call(
        _main_body,
        grid=(NBLK,),
        in_specs=[
            pl.BlockSpec((BE, GW), lambda i: (i, 0)),
            pl.BlockSpec((BE, NBR), lambda i: (i, 0)),
            pl.BlockSpec((NBR, ATOM), lambda i: (0, 0)),
            pl.BlockSpec((NBR, ATOM), lambda i: (0, 0)),
            pl.BlockSpec((NBR, NBR), lambda i: (0, 0)),
            pl.BlockSpec((1, ATOM), lambda i: (0, 0)),
            pl.BlockSpec((1, ATOM), lambda i: (0, 0)),
            pl.BlockSpec((1, ATOM), lambda i: (0, 0)),
            pl.BlockSpec((1, ATOM), lambda i: (0, 0)),
            pl.BlockSpec((1, NBR), lambda i: (0, 0)),
            pl.BlockSpec((1, NBR), lambda i: (0, 0)),
        ],
        out_specs=[
            pl.BlockSpec((BN_ROWS, ATOM), lambda i: (i, 0)),
            pl.BlockSpec((BE, NBR), lambda i: (i, 0)),
            pl.BlockSpec((1, ATOM), lambda i: (0, 0)),
            pl.BlockSpec((1, ATOM), lambda i: (0, 0)),
        ],
        out_shape=[
            jax.ShapeDtypeStruct((N, ATOM), jnp.float32),
            jax.ShapeDtypeStruct((E, NBR), jnp.float32),
            jax.ShapeDtypeStruct((1, ATOM), jnp.float32),
            jax.ShapeDtypeStruct((1, ATOM), jnp.float32),
        ],
    )(g, edge_attr, wcor, wfil, wbnd, scc, shc, scf, shf, scb, shb)


def _final_body(x_ref, nb_ref, sc_ref, sh_ref, o_ref):
    o_ref[...] = _softplus(x_ref[...] + nb_ref[...] * sc_ref[...] + sh_ref[...])


def _final(x, nbr, sc4, sh4):
    return pl.pallas_call(
        _final_body,
        grid=(N // XBLK,),
        in_specs=[
            pl.BlockSpec((XBLK, ATOM), lambda i: (i, 0)),
            pl.BlockSpec((XBLK, ATOM), lambda i: (i, 0)),
            pl.BlockSpec((1, ATOM), lambda i: (0, 0)),
            pl.BlockSpec((1, ATOM), lambda i: (0, 0)),
        ],
        out_specs=pl.BlockSpec((XBLK, ATOM), lambda i: (i, 0)),
        out_shape=jax.ShapeDtypeStruct((N, ATOM), jnp.float32),
    )(x, nbr, sc4, sh4)


def kernel(x, edge_index, edge_attr, W_core, b_core, W_filter, b_filter,
           W_bond, b_bond, g1, be1, g2, be2, g3, be3, g4, be4):
    zpad = jnp.zeros((ATOM, 120), jnp.float32)

    def planes(lohi):
        lo, hi = lohi
        return jnp.concatenate(lo + [zpad] + hi + [zpad], axis=1)

    wdt = planes(([W_core[:, :ATOM].T, W_bond[:, :ATOM].T[:, :8]],
                  [W_filter[:, :ATOM].T, W_bond[:, :ATOM].T[:, 8:]]))
    wst = planes(([W_core[:, ATOM:2 * ATOM].T, W_bond[:, ATOM:2 * ATOM].T[:, :8]],
                  [W_filter[:, ATOM:2 * ATOM].T, W_bond[:, ATOM:2 * ATOM].T[:, 8:]]))
    wcor = W_core[:, 2 * ATOM:].T
    wfil = W_filter[:, 2 * ATOM:].T
    wbnd = W_bond[:, 2 * ATOM:].T
    didx = edge_index[1]
    sidx = edge_index[0]

    p, q = _node_transform(x, wdt, wst)
    p32 = lax.bitcast_convert_type(
        jnp.stack([p[:, :CW], p[:, CW:]], axis=-1), jnp.int32)
    q32 = lax.bitcast_convert_type(
        jnp.stack([q[:, :CW], q[:, CW:]], axis=-1), jnp.int32)
    g = _sc_gather(p32, q32, didx, sidx)

    sc_, qc_, sf_, qf_, sb_, qb_ = _stats(g, edge_attr, wcor, wfil, wbnd)

    def affine(s, q2, gamma, beta, n):
        m = s[0] / n
        v = q2[0] / n - m * m
        scale = gamma * lax.rsqrt(v + 1e-5)
        return scale, beta - m * scale

    scc, shc = affine(sc_, qc_, g1, be1, E)
    scf, shf = affine(sf_, qf_, g2, be2, E)
    scb, shb = affine(sb_, qb_, g3, be3, E)

    nbr, bond_out, nsum, nsq = _main(g, edge_attr, wcor, wfil, wbnd,
                                     scc[None, :], shc[None, :],
                                     scf[None, :], shf[None, :],
                                     scb[None, :], shb[None, :])
    sc4, sh4 = affine(nsum, nsq, g4, be4, N)

    out = _final(x, nbr, sc4[None, :], sh4[None, :])
    return out, bond_out


# speedup vs baseline: 1.1587x; 1.1587x over previous
import jax
import jax.numpy as jnp
from jax import lax
from jax.experimental import pallas as pl
from jax.experimental.pallas import tpu as pltpu
from jax.experimental.pallas import tpu_sc as plsc

N = 10000
NUM_NBR = 32
E = N * NUM_NBR
ATOM = 128
NBR = 16
C = 2 * ATOM + NBR
CW = 256
CP = 2 * CW
HMASK = -65536
RHALF = 0x8000

NC = 2
NS = 16
NW = NC * NS
EPW = E // NW
CB = 40
NCHUNK = EPW // CB

BE = 6400
NBLK = E // BE
BN_ROWS = BE // NUM_NBR
XBLK = 2000


def _softplus(v):
    return jnp.maximum(v, 0.0) + jnp.log1p(jnp.exp(-jnp.abs(v)))


def _sigmoid(v):
    return 1.0 / (1.0 + jnp.exp(-v))


def _node_transform_body(x_ref, wd_ref, ws_ref, p_ref, q_ref):
    xv = x_ref[...]
    p_ref[...] = jnp.dot(xv, wd_ref[...],
                         preferred_element_type=jnp.float32).astype(jnp.bfloat16)
    q_ref[...] = jnp.dot(xv, ws_ref[...],
                         preferred_element_type=jnp.float32).astype(jnp.bfloat16)


def _node_transform(x, wdt, wst):
    return pl.pallas_call(
        _node_transform_body,
        grid=(N // XBLK,),
        in_specs=[
            pl.BlockSpec((XBLK, ATOM), lambda i: (i, 0)),
            pl.BlockSpec((ATOM, CP), lambda i: (0, 0)),
            pl.BlockSpec((ATOM, CP), lambda i: (0, 0)),
        ],
        out_specs=[
            pl.BlockSpec((XBLK, CP), lambda i: (i, 0)),
            pl.BlockSpec((XBLK, CP), lambda i: (i, 0)),
        ],
        out_shape=[jax.ShapeDtypeStruct((N, CP), jnp.bfloat16)] * 2,
    )(x, wdt, wst)


def _sc_gather_body(p_hbm, q_hbm, didx_hbm, sidx_hbm, g_hbm,
                    idxd_v, idxs_v, bufp, bufq,
                    semp0, semp1, semq0, semq1, semw0, semw1):
    wid = lax.axis_index("s") * NC + lax.axis_index("c")
    base = pl.multiple_of(wid * EPW, 8)
    pltpu.sync_copy(didx_hbm.at[pl.ds(base, EPW)], idxd_v)
    pltpu.sync_copy(sidx_hbm.at[pl.ds(base, EPW)], idxs_v)
    semp = (semp0, semp1)
    semq = (semq0, semq1)
    semw = (semw0, semw1)

    def fire(c, b):
        coff = pl.multiple_of(c * CB, 8)
        pltpu.async_copy(p_hbm.at[idxd_v.at[pl.ds(coff, CB)]],
                         bufp.at[b], semp[b])
        pltpu.async_copy(q_hbm.at[idxs_v.at[pl.ds(coff, CB)]],
                         bufq.at[b], semq[b])

    def wait_gathers(b):
        pltpu.make_async_copy(p_hbm.at[idxd_v.at[pl.ds(0, CB)]],
                              bufp.at[b], semp[b]).wait()
        pltpu.make_async_copy(q_hbm.at[idxs_v.at[pl.ds(0, CB)]],
                              bufq.at[b], semq[b]).wait()

    def wait_write(b):
        pltpu.make_async_copy(bufp.at[b], g_hbm.at[pl.ds(0, CB)],
                              semw[b]).wait()

    fire(0, 0)

    def outer(ci2, carry):
        for b in range(2):
            c = ci2 * 2 + b
            nxt_ok = c + 1 < NCHUNK

            @pl.when(jnp.logical_and(nxt_ok, c >= 1))
            def _():
                wait_write(1 - b)

            @pl.when(nxt_ok)
            def _():
                fire(c + 1, 1 - b)

            wait_gathers(b)

            def row(r, cc):
                for j in range(CW // 16):
                    sl = pl.ds(j * 16, 16)
                    wp = bufp[b, r, sl]
                    wq = bufq[b, r, sl]
                    bcf = lambda v: lax.bitcast_convert_type(v, jnp.float32)
                    bci = lambda v: lax.bitcast_convert_type(v, jnp.int32)
                    lo = (bcf(lax.shift_left(wp, 16))
                          + bcf(lax.shift_left(wq, 16)))
                    lor = lax.shift_right_logical(bci(lo) + RHALF, 16)
                    if j == 0:
                        hi = bcf(wp & HMASK) + bcf(wq & HMASK)
                        hir = (bci(hi) + RHALF) & HMASK
                        bufp[b, r, sl] = hir | lor
                    else:
                        bufp[b, r, sl] = lor
                return cc

            lax.fori_loop(0, CB, row, 0)
            off = pl.multiple_of(base + c * CB, 8)
            pltpu.async_copy(bufp.at[b], g_hbm.at[pl.ds(off, CB)], semw[b])
        return carry

    lax.fori_loop(0, NCHUNK // 2, outer, 0)
    wait_write(0)
    wait_write(1)


def _sc_gather(p, q, didx, sidx):
    mesh = plsc.VectorSubcoreMesh(core_axis_name="c", subcore_axis_name="s")
    fn = pl.kernel(
        _sc_gather_body,
        mesh=mesh,
        out_type=jax.ShapeDtypeStruct((E, CW), jnp.int32),
        scratch_types=[
            pltpu.VMEM((EPW,), jnp.int32),
            pltpu.VMEM((EPW,), jnp.int32),
            pltpu.VMEM((2, CB, CW), jnp.int32),
            pltpu.VMEM((2, CB, CW), jnp.int32),
            pltpu.SemaphoreType.DMA,
            pltpu.SemaphoreType.DMA,
            pltpu.SemaphoreType.DMA,
            pltpu.SemaphoreType.DMA,
            pltpu.SemaphoreType.DMA,
            pltpu.SemaphoreType.DMA,
        ],
    )
    return fn(p, q, didx, sidx)


def _stats_body(g_ref, a_ref, wlo_ref, whi_ref,
                slo_ref, qlo_ref, shi_ref, qhi_ref):
    a = a_ref[...]
    gi = g_ref[...]
    f_lo = lax.bitcast_convert_type(lax.shift_left(gi, 16), jnp.float32)
    f_hi = lax.bitcast_convert_type(gi[:, :NBR] & HMASK, jnp.float32)
    h_lo = f_lo + jnp.dot(a, wlo_ref[...], preferred_element_type=jnp.float32)
    h_hi = f_hi + jnp.dot(a, whi_ref[...], preferred_element_type=jnp.float32)

    @pl.when(pl.program_id(0) == 0)
    def _():
        slo_ref[...] = jnp.zeros_like(slo_ref)
        qlo_ref[...] = jnp.zeros_like(qlo_ref)
        shi_ref[...] = jnp.zeros_like(shi_ref)
        qhi_ref[...] = jnp.zeros_like(qhi_ref)

    slo_ref[...] += jnp.sum(h_lo, axis=0, keepdims=True)
    qlo_ref[...] += jnp.sum(h_lo * h_lo, axis=0, keepdims=True)
    shi_ref[...] += jnp.sum(h_hi, axis=0, keepdims=True)
    qhi_ref[...] += jnp.sum(h_hi * h_hi, axis=0, keepdims=True)


def _stats(g, edge_attr, wlo, whi):
    return pl.pallas_call(
        _stats_body,
        grid=(NBLK,),
        in_specs=[
            pl.BlockSpec((BE, CW), lambda i: (i, 0)),
            pl.BlockSpec((BE, NBR), lambda i: (i, 0)),
            pl.BlockSpec((NBR, CW), lambda i: (0, 0)),
            pl.BlockSpec((NBR, NBR), lambda i: (0, 0)),
        ],
        out_specs=[
            pl.BlockSpec((1, CW), lambda i: (0, 0)),
            pl.BlockSpec((1, CW), lambda i: (0, 0)),
            pl.BlockSpec((1, NBR), lambda i: (0, 0)),
            pl.BlockSpec((1, NBR), lambda i: (0, 0)),
        ],
        out_shape=[
            jax.ShapeDtypeStruct((1, CW), jnp.float32),
            jax.ShapeDtypeStruct((1, CW), jnp.float32),
            jax.ShapeDtypeStruct((1, NBR), jnp.float32),
            jax.ShapeDtypeStruct((1, NBR), jnp.float32),
        ],
    )(g, edge_attr, wlo, whi)


def _main_body(g_ref, a_ref, wlo_ref, whi_ref, sclo_ref, shlo_ref,
               schi_ref, shhi_ref, nbr_ref, bond_ref, s_ref, sq_ref):
    a = a_ref[...]
    gi = g_ref[...]
    f_lo = lax.bitcast_convert_type(lax.shift_left(gi, 16), jnp.float32)
    f_hi = lax.bitcast_convert_type(gi[:, :NBR] & HMASK, jnp.float32)
    h_lo = f_lo + jnp.dot(a, wlo_ref[...], preferred_element_type=jnp.float32)
    h_hi = f_hi + jnp.dot(a, whi_ref[...], preferred_element_type=jnp.float32)
    y_lo = h_lo * sclo_ref[...] + shlo_ref[...]
    af = _sigmoid(y_lo[:, :ATOM])
    ac = _softplus(y_lo[:, ATOM:])
    prod = af * ac
    nb = prod.reshape(BN_ROWS, NUM_NBR, ATOM).sum(axis=1)
    nbr_ref[...] = nb
    bond_ref[...] = _softplus(a + (h_hi * schi_ref[...] + shhi_ref[...]))

    @pl.when(pl.program_id(0) == 0)
    def _():
        s_ref[...] = jnp.zeros_like(s_ref)
        sq_ref[...] = jnp.zeros_like(sq_ref)

    s_ref[...] += jnp.sum(nb, axis=0, keepdims=True)
    sq_ref[...] += jnp.sum(nb * nb, axis=0, keepdims=True)


def _main(g, edge_attr, wlo, whi, sclo, shlo, schi, shhi):
    return pl.pallas_call(
        _main_body,
        grid=(NBLK,),
        in_specs=[
            pl.BlockSpec((BE, CW), lambda i: (i, 0)),
            pl.BlockSpec((BE, NBR), lambda i: (i, 0)),
            pl.BlockSpec((NBR, CW), lambda i: (0, 0)),
            pl.BlockSpec((NBR, NBR), lambda i: (0, 0)),
            pl.BlockSpec((1, CW), lambda i: (0, 0)),
            pl.BlockSpec((1, CW), lambda i: (0, 0)),
            pl.BlockSpec((1, NBR), lambda i: (0, 0)),
            pl.BlockSpec((1, NBR), lambda i: (0, 0)),
        ],
        out_specs=[
            pl.BlockSpec((BN_ROWS, ATOM), lambda i: (i, 0)),
            pl.BlockSpec((BE, NBR), lambda i: (i, 0)),
            pl.BlockSpec((1, ATOM), lambda i: (0, 0)),
            pl.BlockSpec((1, ATOM), lambda i: (0, 0)),
        ],
        out_shape=[
            jax.ShapeDtypeStruct((N, ATOM), jnp.float32),
            jax.ShapeDtypeStruct((E, NBR), jnp.float32),
            jax.ShapeDtypeStruct((1, ATOM), jnp.float32),
            jax.ShapeDtypeStruct((1, ATOM), jnp.float32),
        ],
    )(g, edge_attr, wlo, whi, sclo, shlo, schi, shhi)


def _final_body(x_ref, nb_ref, sc_ref, sh_ref, o_ref):
    o_ref[...] = _softplus(x_ref[...] + nb_ref[...] * sc_ref[...] + sh_ref[...])


def _final(x, nbr, sc4, sh4):
    return pl.pallas_call(
        _final_body,
        grid=(N // XBLK,),
        in_specs=[
            pl.BlockSpec((XBLK, ATOM), lambda i: (i, 0)),
            pl.BlockSpec((XBLK, ATOM), lambda i: (i, 0)),
            pl.BlockSpec((1, ATOM), lambda i: (0, 0)),
            pl.BlockSpec((1, ATOM), lambda i: (0, 0)),
        ],
        out_specs=pl.BlockSpec((XBLK, ATOM), lambda i: (i, 0)),
        out_shape=jax.ShapeDtypeStruct((N, ATOM), jnp.float32),
    )(x, nbr, sc4, sh4)


def kernel(x, edge_index, edge_attr, W_core, b_core, W_filter, b_filter,
           W_bond, b_bond, g1, be1, g2, be2, g3, be3, g4, be4):
    ws = (W_core, W_filter, W_bond)
    pad_w = jnp.zeros((ATOM, CP - C), jnp.float32)
    wdt = jnp.concatenate([w[:, :ATOM].T for w in ws] + [pad_w], axis=1)
    wst = jnp.concatenate([w[:, ATOM:2 * ATOM].T for w in ws] + [pad_w], axis=1)
    wlo = jnp.concatenate([W_core[:, 2 * ATOM:].T,
                           W_filter[:, 2 * ATOM:].T], axis=1)
    whi = W_bond[:, 2 * ATOM:].T
    didx = edge_index[1]
    sidx = edge_index[0]

    p, q = _node_transform(x, wdt, wst)
    p32 = lax.bitcast_convert_type(
        jnp.stack([p[:, :CW], p[:, CW:]], axis=-1), jnp.int32)
    q32 = lax.bitcast_convert_type(
        jnp.stack([q[:, :CW], q[:, CW:]], axis=-1), jnp.int32)
    g = _sc_gather(p32, q32, didx, sidx)

    slo, qlo, shi, qhi = _stats(g, edge_attr, wlo, whi)
    glo = jnp.concatenate([g1, g2])
    blo = jnp.concatenate([be1, be2])
    mlo = slo[0] / E
    vlo = qlo[0] / E - mlo * mlo
    sclo = glo * lax.rsqrt(vlo + 1e-5)
    shlo = blo - mlo * sclo
    mhi = shi[0] / E
    vhi = qhi[0] / E - mhi * mhi
    schi = g3 * lax.rsqrt(vhi + 1e-5)
    shhi = be3 - mhi * schi

    nbr, bond_out, nsum, nsq = _main(g, edge_attr, wlo, whi,
                                     sclo[None, :], shlo[None, :],
                                     schi[None, :], shhi[None, :])
    m4 = nsum[0] / N
    v4 = nsq[0] / N - m4 * m4
    sc4 = g4 * lax.rsqrt(v4 + 1e-5)
    sh4 = be4 - m4 * sc4

    out = _final(x, nbr, sc4[None, :], sh4[None, :])
    return out, bond_out
